# Initial kernel scaffold; baseline (speedup 1.0000x reference)
import jax
import jax.numpy as jnp
from jax.experimental import pallas as pl


def kernel(x, table, coeffs0, coeffs1):
    raise NotImplementedError("write your pallas kernel here")


# speedup vs baseline: 7.5817x; 7.5817x over previous
import jax
import jax.numpy as jnp
from jax import lax
from jax.experimental import pallas as pl
from jax.experimental.pallas import tpu as pltpu
from jax.experimental.pallas import tpu_sc as plsc

SIZE = 8388608
DIM = 64
N_CHUNKS = 8
BATCH = 16384
OUT_RANGE = SIZE // 2
MASK22 = OUT_RANGE - 1
MERSENNE = (1 << 31) - 1

NC, NS = 2, 16
NW = NC * NS
B_PER_W = BATCH // NW
BLK = 32
NBLK = B_PER_W // BLK
CPB = BLK * N_CHUNKS
SROWS = 2 * CPB
ROWS = SROWS + CPB
SCALE8 = (N_CHUNKS * DIM) ** 0.5 / N_CHUNKS


def _iota16():
    return lax.iota(jnp.int32, 16)


def _bcast(scalar):
    return jnp.full((16,), scalar, dtype=jnp.int32)


def _can(v):
    w = lax.shift_right_logical(v, jnp.int32(31)) + (v & jnp.int32(0x7FFFFFFF))
    w2 = w - jnp.int32(MERSENNE)
    return jnp.where(w2 >= 0, w2, w)


def _hash_pair(xl, xh, a, b):
    al = a & jnp.int32(0xFFFF)
    ah = lax.shift_right_logical(a, jnp.int32(16))
    p0 = xl * al
    t = xh * al + xl * ah
    term1 = (xh * ah) * jnp.int32(2)
    t2 = _can(t)
    mid = lax.shift_right_logical(t2, jnp.int32(15)) + lax.shift_left(
        t2 & jnp.int32(0x7FFF), jnp.int32(16))
    s = _can(term1 + mid)
    s = _can(s + _can(p0))
    return _can(s + b)


def _body(x_hbm, tab_hbm, cf_hbm, out_hbm,
          x_v, cf_v, ridx_v, off_v, woff_v, rows_v, out_v, sem_r):
    wid = lax.axis_index("s") * NC + lax.axis_index("c")
    pltpu.sync_copy(cf_hbm, cf_v)

    def block(g, _):
        base = wid * B_PER_W + g * BLK
        pltpu.sync_copy(x_hbm.at[pl.ds(base, BLK)], x_v)

        for grp in range(BLK // 16):
            x_vec = x_v[pl.ds(grp * 16, 16)]
            xl = x_vec & jnp.int32(0xFFFF)
            xh = lax.shift_right_logical(x_vec, jnp.int32(16))
            lanes = _iota16()
            for c in range(N_CHUNKS):
                a0 = cf_v[jnp.int32(c), :]
                b0 = cf_v[jnp.int32(8 + c), :]
                a1 = cf_v[jnp.int32(16 + c), :]
                b1 = cf_v[jnp.int32(24 + c), :]
                h0 = _hash_pair(xl, xh, a0, b0) & jnp.int32(MASK22)
                h1 = _hash_pair(xl, xh, a1, b1) & jnp.int32(MASK22)
                r = lax.shift_right_logical(h0, jnp.int32(6))
                o = h0 & jnp.int32(63)
                r1 = lax.shift_right_logical(h1, jnp.int32(6))
                o1 = h1 & jnp.int32(63)
                wpos = (grp * 16 + lanes) * jnp.int32(N_CHUNKS) + jnp.int32(c)
                pos = wpos * jnp.int32(2)
                prow = lax.shift_right_logical(pos, jnp.int32(7))
                pcol = pos & jnp.int32(127)
                plsc.store_scatter(ridx_v, [prow, pcol], r)
                plsc.store_scatter(ridx_v, [prow, pcol + 1],
                                   r + jnp.int32(1))
                qpos = wpos + jnp.int32(SROWS)
                qrow = lax.shift_right_logical(qpos, jnp.int32(7))
                qcol = qpos & jnp.int32(127)
                plsc.store_scatter(ridx_v, [qrow, qcol], r1)
                plsc.store_scatter(off_v, [wpos], o)
                plsc.store_scatter(woff_v, [wpos], o1)

        cps = [
            pltpu.async_copy(tab_hbm.at[ridx_v.at[jnp.int32(j)]],
                             rows_v.at[pl.ds(j * 128, 128)], sem_r)
            for j in range(ROWS // 128)
        ]
        for cp in cps:
            cp.wait()

        def combine(b, _):
            accs = [jnp.zeros((16,), jnp.float32) for _ in range(4)]
            for c in range(N_CHUNKS):
                bc = b * N_CHUNKS + c
                wcol = plsc.load_gather(woff_v, [_bcast(bc)])
                wgt = plsc.load_gather(rows_v, [_bcast(SROWS + bc), wcol])
                off = plsc.load_gather(off_v, [_bcast(bc)])
                fr0 = _bcast(2 * bc)
                for k in range(4):
                    w_idx = off + (_iota16() + jnp.int32(16 * k))
                    fr = fr0 + lax.shift_right_logical(w_idx, jnp.int32(6))
                    col = w_idx & jnp.int32(63)
                    v = plsc.load_gather(rows_v, [fr, col])
                    accs[k] = accs[k] + v * wgt
            for k in range(4):
                out_v[b, pl.ds(16 * k, 16)] = accs[k] * jnp.float32(SCALE8)
            return _

        lax.fori_loop(jnp.int32(0), jnp.int32(BLK), combine, None)
        pltpu.sync_copy(out_v, out_hbm.at[pl.ds(base, BLK), :])
        return _

    lax.fori_loop(jnp.int32(0), jnp.int32(NBLK), block, None)


@jax.jit
def _run(x32, table2, cf32):
    mesh = plsc.VectorSubcoreMesh(core_axis_name="c", subcore_axis_name="s")
    kfn = pl.kernel(
        _body,
        out_type=jax.ShapeDtypeStruct((BATCH, DIM), jnp.float32),
        mesh=mesh,
        scratch_types=[
            pltpu.VMEM((BLK,), jnp.int32),
            pltpu.VMEM((32, 16), jnp.int32),
            pltpu.VMEM((ROWS // 128, 128), jnp.int32),
            pltpu.VMEM((CPB,), jnp.int32),
            pltpu.VMEM((CPB,), jnp.int32),
            pltpu.VMEM((ROWS, DIM), jnp.float32),
            pltpu.VMEM((BLK, DIM), jnp.float32),
            pltpu.SemaphoreType.DMA,
        ],
        compiler_params=pltpu.CompilerParams(
            needs_layout_passes=False, use_tc_tiling_on_sc=False),
    )
    return kfn(x32, table2, cf32)


def kernel(x, table, coeffs0, coeffs1):
    x32 = x.astype(jnp.int32)
    cf32 = jnp.concatenate([
        coeffs0[:, 0], coeffs0[:, 1], coeffs1[:, 0], coeffs1[:, 1],
    ]).astype(jnp.int32)
    cf32 = jnp.tile(cf32[:, None], (1, 16))
    table2 = table.reshape(SIZE // DIM, DIM)
    return _run(x32, table2, cf32)


# speedup vs baseline: 11.2010x; 1.4774x over previous
import jax
import jax.numpy as jnp
from jax import lax
from jax.experimental import pallas as pl
from jax.experimental.pallas import tpu as pltpu
from jax.experimental.pallas import tpu_sc as plsc

SIZE = 8388608
DIM = 64
N_CHUNKS = 8
BATCH = 16384
OUT_RANGE = SIZE // 2
MASK22 = OUT_RANGE - 1
MERSENNE = (1 << 31) - 1

NC, NS = 2, 16
NW = NC * NS
B_PER_W = BATCH // NW
BLK = 32
NBLK = B_PER_W // BLK
CPB = BLK * N_CHUNKS
SROWS = 2 * CPB
ROWS = SROWS + CPB
SCALE8 = (N_CHUNKS * DIM) ** 0.5 / N_CHUNKS


def _iota16():
    return lax.iota(jnp.int32, 16)


def _bcast(scalar):
    return jnp.full((16,), scalar, dtype=jnp.int32)


def _can(v):
    w = lax.shift_right_logical(v, jnp.int32(31)) + (v & jnp.int32(0x7FFFFFFF))
    w2 = w - jnp.int32(MERSENNE)
    return jnp.where(w2 >= 0, w2, w)


def _hash_pair(xl, xh, a, b):
    al = a & jnp.int32(0xFFFF)
    ah = lax.shift_right_logical(a, jnp.int32(16))
    p0 = xl * al
    t = xh * al + xl * ah
    term1 = (xh * ah) * jnp.int32(2)
    t2 = _can(t)
    mid = lax.shift_right_logical(t2, jnp.int32(15)) + lax.shift_left(
        t2 & jnp.int32(0x7FFF), jnp.int32(16))
    s = _can(term1 + mid)
    s = _can(s + _can(p0))
    return _can(s + b)


def _body(x_hbm, tab_hbm, cf_hbm, out_hbm,
          x_v, cf_v, ridx_v, pk_v, rows_v, out_v, sem_r):
    wid = lax.axis_index("s") * NC + lax.axis_index("c")
    pltpu.sync_copy(cf_hbm, cf_v)

    def hash_and_fire(gg):
        buf = gg & jnp.int32(1)
        base = wid * B_PER_W + gg * BLK
        pltpu.sync_copy(x_hbm.at[pl.ds(base, BLK)], x_v)
        buf6 = buf * jnp.int32(6)
        buf256 = buf * jnp.int32(CPB)
        for grp in range(BLK // 16):
            x_vec = x_v[pl.ds(grp * 16, 16)]
            xl = x_vec & jnp.int32(0xFFFF)
            xh = lax.shift_right_logical(x_vec, jnp.int32(16))
            lanes = _iota16()
            for c in range(N_CHUNKS):
                a0 = cf_v[jnp.int32(c), :]
                b0 = cf_v[jnp.int32(8 + c), :]
                a1 = cf_v[jnp.int32(16 + c), :]
                b1 = cf_v[jnp.int32(24 + c), :]
                h0 = _hash_pair(xl, xh, a0, b0) & jnp.int32(MASK22)
                h1 = _hash_pair(xl, xh, a1, b1) & jnp.int32(MASK22)
                r = lax.shift_right_logical(h0, jnp.int32(6))
                o = h0 & jnp.int32(63)
                r1 = lax.shift_right_logical(h1, jnp.int32(6))
                o1 = h1 & jnp.int32(63)
                wpos = (grp * 16 + lanes) * jnp.int32(N_CHUNKS) + jnp.int32(c)
                pos = wpos * jnp.int32(2)
                prow = buf6 + lax.shift_right_logical(pos, jnp.int32(7))
                pcol = pos & jnp.int32(127)
                plsc.store_scatter(ridx_v, [prow, pcol], r)
                plsc.store_scatter(ridx_v, [prow, pcol + 1],
                                   r + jnp.int32(1))
                qpos = wpos + jnp.int32(SROWS)
                qrow = buf6 + lax.shift_right_logical(qpos, jnp.int32(7))
                qcol = qpos & jnp.int32(127)
                plsc.store_scatter(ridx_v, [qrow, qcol], r1)
                pk = o + lax.shift_left(o1, jnp.int32(16))
                plsc.store_scatter(pk_v, [buf256 + wpos], pk)
        for j in range(ROWS // 128):
            pltpu.async_copy(
                tab_hbm.at[ridx_v.at[buf6 + jnp.int32(j)]],
                rows_v.at[pl.ds(buf * ROWS + j * 128, 128)], sem_r)

    def block(g, _):
        buf = g & jnp.int32(1)

        @pl.when(g < jnp.int32(NBLK - 1))
        def _fire_next():
            hash_and_fire(g + jnp.int32(1))

        pltpu.make_async_copy(
            tab_hbm.at[pl.ds(0, ROWS)],
            rows_v.at[pl.ds(buf * ROWS, ROWS)], sem_r).wait()

        buf768 = buf * jnp.int32(ROWS)
        buf256 = buf * jnp.int32(CPB)
        iks = [_iota16() + jnp.int32(16 * k) for k in range(4)]

        @plsc.parallel_loop(jnp.int32(0), jnp.int32(BLK), jnp.int32(1))
        def combine(b):
            accs = [jnp.zeros((16,), jnp.float32) for _ in range(4)]
            for c in range(N_CHUNKS):
                bc = b * N_CHUNKS + c
                pk = plsc.load_gather(pk_v, [_bcast(buf256 + bc)])
                off = pk & jnp.int32(0xFFFF)
                wcol = lax.shift_right_logical(pk, jnp.int32(16))
                wgt = plsc.load_gather(
                    rows_v, [_bcast(buf768 + SROWS + bc), wcol])
                fr0 = _bcast(buf768 + 2 * bc)
                for k in range(4):
                    w_idx = off + iks[k]
                    fr = fr0 + lax.shift_right_logical(w_idx, jnp.int32(6))
                    col = w_idx & jnp.int32(63)
                    v = plsc.load_gather(rows_v, [fr, col])
                    accs[k] = accs[k] + v * wgt
            for k in range(4):
                out_v[b, pl.ds(16 * k, 16)] = accs[k] * jnp.float32(SCALE8)

        base = wid * B_PER_W + g * BLK
        pltpu.sync_copy(out_v, out_hbm.at[pl.ds(base, BLK), :])
        return _

    hash_and_fire(jnp.int32(0))
    lax.fori_loop(jnp.int32(0), jnp.int32(NBLK), block, None)


@jax.jit
def _run(x32, table2, cf32):
    mesh = plsc.VectorSubcoreMesh(core_axis_name="c", subcore_axis_name="s")
    kfn = pl.kernel(
        _body,
        out_type=jax.ShapeDtypeStruct((BATCH, DIM), jnp.float32),
        mesh=mesh,
        scratch_types=[
            pltpu.VMEM((BLK,), jnp.int32),
            pltpu.VMEM((32, 16), jnp.int32),
            pltpu.VMEM((2 * ROWS // 128, 128), jnp.int32),
            pltpu.VMEM((2 * CPB,), jnp.int32),
            pltpu.VMEM((2 * ROWS, DIM), jnp.float32),
            pltpu.VMEM((BLK, DIM), jnp.float32),
            pltpu.SemaphoreType.DMA,
        ],
        compiler_params=pltpu.CompilerParams(
            needs_layout_passes=False, use_tc_tiling_on_sc=False),
    )
    return kfn(x32, table2, cf32)


def kernel(x, table, coeffs0, coeffs1):
    x32 = x.astype(jnp.int32)
    cf32 = jnp.concatenate([
        coeffs0[:, 0], coeffs0[:, 1], coeffs1[:, 0], coeffs1[:, 1],
    ]).astype(jnp.int32)
    cf32 = jnp.tile(cf32[:, None], (1, 16))
    table2 = table.reshape(SIZE // DIM, DIM)
    return _run(x32, table2, cf32)


# speedup vs baseline: 11.5987x; 1.0355x over previous
import jax
import jax.numpy as jnp
from jax import lax
from jax.experimental import pallas as pl
from jax.experimental.pallas import tpu as pltpu
from jax.experimental.pallas import tpu_sc as plsc

SIZE = 8388608
DIM = 64
N_CHUNKS = 8
BATCH = 16384
OUT_RANGE = SIZE // 2
MASK22 = OUT_RANGE - 1
MERSENNE = (1 << 31) - 1

NC, NS = 2, 16
NW = NC * NS
B_PER_W = BATCH // NW
BLK = 32
NBLK = B_PER_W // BLK
CPB = BLK * N_CHUNKS
SROWS = 2 * CPB
ROWS = SROWS + CPB
SCALE8 = (N_CHUNKS * DIM) ** 0.5 / N_CHUNKS


def _iota16():
    return lax.iota(jnp.int32, 16)


def _bcast(scalar):
    return jnp.full((16,), scalar, dtype=jnp.int32)


def _can(v):
    w = lax.shift_right_logical(v, jnp.int32(31)) + (v & jnp.int32(0x7FFFFFFF))
    w2 = w - jnp.int32(MERSENNE)
    return jnp.where(w2 >= 0, w2, w)


def _hash_pair(xl, xh, a, b):
    al = a & jnp.int32(0xFFFF)
    ah = lax.shift_right_logical(a, jnp.int32(16))
    p0 = xl * al
    t = xh * al + xl * ah
    term1 = (xh * ah) * jnp.int32(2)
    t2 = _can(t)
    mid = lax.shift_right_logical(t2, jnp.int32(15)) + lax.shift_left(
        t2 & jnp.int32(0x7FFF), jnp.int32(16))
    s = _can(term1 + mid)
    s = _can(s + _can(p0))
    return _can(s + b)


def _body(x_hbm, tab_hbm, cf_hbm, out_hbm,
          x_v, cf_v, ridx_v, pk_v, rows_v, out_v, sem_r):
    wid = lax.axis_index("s") * NC + lax.axis_index("c")
    pltpu.sync_copy(cf_hbm, cf_v)

    def hash_and_fire(gg):
        buf = gg & jnp.int32(1)
        base = wid * B_PER_W + gg * BLK
        pltpu.sync_copy(x_hbm.at[pl.ds(base, BLK)], x_v)
        buf6 = buf * jnp.int32(6)
        buf256 = buf * jnp.int32(CPB)
        for grp in range(BLK // 16):
            x_vec = x_v[pl.ds(grp * 16, 16)]
            xl = x_vec & jnp.int32(0xFFFF)
            xh = lax.shift_right_logical(x_vec, jnp.int32(16))
            lanes = _iota16()
            for c in range(N_CHUNKS):
                a0 = cf_v[jnp.int32(c), :]
                b0 = cf_v[jnp.int32(8 + c), :]
                a1 = cf_v[jnp.int32(16 + c), :]
                b1 = cf_v[jnp.int32(24 + c), :]
                h0 = _hash_pair(xl, xh, a0, b0) & jnp.int32(MASK22)
                h1 = _hash_pair(xl, xh, a1, b1) & jnp.int32(MASK22)
                r = lax.shift_right_logical(h0, jnp.int32(6))
                o = h0 & jnp.int32(63)
                r1 = lax.shift_right_logical(h1, jnp.int32(6))
                o1 = h1 & jnp.int32(63)
                wpos = (grp * 16 + lanes) * jnp.int32(N_CHUNKS) + jnp.int32(c)
                pos = wpos * jnp.int32(2)
                prow = buf6 + lax.shift_right_logical(pos, jnp.int32(7))
                pcol = pos & jnp.int32(127)
                plsc.store_scatter(ridx_v, [prow, pcol], r)
                plsc.store_scatter(ridx_v, [prow, pcol + 1],
                                   r + jnp.int32(1))
                qpos = wpos + jnp.int32(SROWS)
                qrow = buf6 + lax.shift_right_logical(qpos, jnp.int32(7))
                qcol = qpos & jnp.int32(127)
                plsc.store_scatter(ridx_v, [qrow, qcol], r1)
                pk = o + lax.shift_left(o1, jnp.int32(16))
                plsc.store_scatter(pk_v, [buf256 + wpos], pk)
        for j in range(ROWS // 128):
            pltpu.async_copy(
                tab_hbm.at[ridx_v.at[buf6 + jnp.int32(j)]],
                rows_v.at[pl.ds(buf * ROWS + j * 128, 128)], sem_r)

    def block(g, _):
        buf = g & jnp.int32(1)

        @pl.when(g < jnp.int32(NBLK - 1))
        def _fire_next():
            hash_and_fire(g + jnp.int32(1))

        pltpu.make_async_copy(
            tab_hbm.at[pl.ds(0, ROWS)],
            rows_v.at[pl.ds(buf * ROWS, ROWS)], sem_r).wait()

        buf768 = buf * jnp.int32(ROWS)
        buf256 = buf * jnp.int32(CPB)
        iks = [_iota16() + jnp.int32(16 * k) for k in range(4)]
        zero16 = _bcast(jnp.int32(0) * buf)
        wbase0 = (buf768 + jnp.int32(SROWS)) * jnp.int32(DIM)

        @plsc.parallel_loop(jnp.int32(0), jnp.int32(BLK), jnp.int32(1))
        def combine(b):
            accs = [jnp.zeros((16,), jnp.float32) for _ in range(4)]
            for c in range(N_CHUNKS):
                bc = b * N_CHUNKS + c
                pk = plsc.load_gather(pk_v, [_bcast(buf256 + bc)])
                off = pk & jnp.int32(0xFFFF)
                wflat = _bcast(wbase0 + bc * jnp.int32(DIM))                     + lax.shift_right_logical(pk, jnp.int32(16))
                wgt = plsc.load_gather(rows_v, [zero16, wflat])
                sbase = _bcast((buf768 + 2 * bc) * jnp.int32(DIM)) + off
                for k in range(4):
                    v = plsc.load_gather(rows_v, [zero16, sbase + iks[k]])
                    accs[k] = accs[k] + v * wgt
            for k in range(4):
                out_v[b, pl.ds(16 * k, 16)] = accs[k] * jnp.float32(SCALE8)

        base = wid * B_PER_W + g * BLK
        pltpu.sync_copy(out_v, out_hbm.at[pl.ds(base, BLK), :])
        return _

    hash_and_fire(jnp.int32(0))
    lax.fori_loop(jnp.int32(0), jnp.int32(NBLK), block, None)


@jax.jit
def _run(x32, table2, cf32):
    mesh = plsc.VectorSubcoreMesh(core_axis_name="c", subcore_axis_name="s")
    kfn = pl.kernel(
        _body,
        out_type=jax.ShapeDtypeStruct((BATCH, DIM), jnp.float32),
        mesh=mesh,
        scratch_types=[
            pltpu.VMEM((BLK,), jnp.int32),
            pltpu.VMEM((32, 16), jnp.int32),
            pltpu.VMEM((2 * ROWS // 128, 128), jnp.int32),
            pltpu.VMEM((2 * CPB,), jnp.int32),
            pltpu.VMEM((2 * ROWS, DIM), jnp.float32),
            pltpu.VMEM((BLK, DIM), jnp.float32),
            pltpu.SemaphoreType.DMA,
        ],
        compiler_params=pltpu.CompilerParams(
            needs_layout_passes=False, use_tc_tiling_on_sc=False),
    )
    return kfn(x32, table2, cf32)


def kernel(x, table, coeffs0, coeffs1):
    x32 = x.astype(jnp.int32)
    cf32 = jnp.concatenate([
        coeffs0[:, 0], coeffs0[:, 1], coeffs1[:, 0], coeffs1[:, 1],
    ]).astype(jnp.int32)
    cf32 = jnp.tile(cf32[:, None], (1, 16))
    table2 = table.reshape(SIZE // DIM, DIM)
    return _run(x32, table2, cf32)


# speedup vs baseline: 11.9131x; 1.0271x over previous
import jax
import jax.numpy as jnp
from jax import lax
from jax.experimental import pallas as pl
from jax.experimental.pallas import tpu as pltpu
from jax.experimental.pallas import tpu_sc as plsc

SIZE = 8388608
DIM = 64
N_CHUNKS = 8
BATCH = 16384
OUT_RANGE = SIZE // 2
MASK22 = OUT_RANGE - 1
MERSENNE = (1 << 31) - 1

NC, NS = 2, 16
NW = NC * NS
B_PER_W = BATCH // NW
BLK = 32
NBLK = B_PER_W // BLK
CPB = BLK * N_CHUNKS
SROWS = 2 * CPB
ROWS = SROWS + CPB
SCALE8 = (N_CHUNKS * DIM) ** 0.5 / N_CHUNKS


def _iota16():
    return lax.iota(jnp.int32, 16)


def _bcast(scalar):
    return jnp.full((16,), scalar, dtype=jnp.int32)


def _can(v):
    w = lax.shift_right_logical(v, jnp.int32(31)) + (v & jnp.int32(0x7FFFFFFF))
    w2 = w - jnp.int32(MERSENNE)
    return jnp.where(w2 >= 0, w2, w)


def _red(v):
    return lax.shift_right_logical(v, jnp.int32(31)) + (v & jnp.int32(0x7FFFFFFF))


def _hash_pair(xl, xh, a, b):
    al = a & jnp.int32(0xFFFF)
    ah = lax.shift_right_logical(a, jnp.int32(16))
    p0 = xl * al
    t = xh * al + xl * ah
    term1 = (xh * ah) * jnp.int32(2)
    t2 = _red(t)
    mid = lax.shift_right_logical(t2, jnp.int32(15)) + lax.shift_left(
        t2 & jnp.int32(0x7FFF), jnp.int32(16))
    m1 = _red(mid + term1)
    m2 = _red(m1 + _red(p0))
    sfull = m2 + b
    w = _red(sfull)
    w2 = w - jnp.int32(MERSENNE)
    return jnp.where(w2 >= 0, w2, w)


def _body(x_hbm, tab_hbm, cf_hbm, out_hbm,
          x_v, cf_v, ridx_v, pk_v, rows_v, out_v, sem_r):
    wid = lax.axis_index("s") * NC + lax.axis_index("c")
    pltpu.sync_copy(cf_hbm, cf_v)

    def hash_and_fire(gg):
        buf = gg & jnp.int32(1)
        base = wid * B_PER_W + gg * BLK
        pltpu.sync_copy(x_hbm.at[pl.ds(base, BLK)], x_v)
        buf6 = buf * jnp.int32(6)
        buf256 = buf * jnp.int32(CPB)
        lanes = _iota16()
        xls, xhs = [], []
        for grp in range(BLK // 16):
            x_vec = x_v[pl.ds(grp * 16, 16)]
            xls.append(x_vec & jnp.int32(0xFFFF))
            xhs.append(lax.shift_right_logical(x_vec, jnp.int32(16)))
        for c in range(N_CHUNKS):
            a0 = cf_v[jnp.int32(c), :]
            b0 = cf_v[jnp.int32(8 + c), :]
            a1 = cf_v[jnp.int32(16 + c), :]
            b1 = cf_v[jnp.int32(24 + c), :]
            for grp in range(BLK // 16):
                xl, xh = xls[grp], xhs[grp]
                h0 = _hash_pair(xl, xh, a0, b0) & jnp.int32(MASK22)
                h1 = _hash_pair(xl, xh, a1, b1) & jnp.int32(MASK22)
                r = lax.shift_right_logical(h0, jnp.int32(6))
                o = h0 & jnp.int32(63)
                r1 = lax.shift_right_logical(h1, jnp.int32(6))
                o1 = h1 & jnp.int32(63)
                wpos = (grp * 16 + lanes) * jnp.int32(N_CHUNKS) + jnp.int32(c)
                pos = wpos * jnp.int32(2)
                prow = buf6 + lax.shift_right_logical(pos, jnp.int32(7))
                pcol = pos & jnp.int32(127)
                plsc.store_scatter(ridx_v, [prow, pcol], r)
                plsc.store_scatter(ridx_v, [prow, pcol + 1],
                                   r + jnp.int32(1))
                qpos = wpos + jnp.int32(SROWS)
                qrow = buf6 + lax.shift_right_logical(qpos, jnp.int32(7))
                qcol = qpos & jnp.int32(127)
                plsc.store_scatter(ridx_v, [qrow, qcol], r1)
                pk = o + lax.shift_left(o1, jnp.int32(16))
                plsc.store_scatter(pk_v, [buf256 + wpos], pk)
        for j in range(ROWS // 128):
            pltpu.async_copy(
                tab_hbm.at[ridx_v.at[buf6 + jnp.int32(j)]],
                rows_v.at[pl.ds(buf * ROWS + j * 128, 128)], sem_r)

    def block(g, _):
        buf = g & jnp.int32(1)

        @pl.when(g < jnp.int32(NBLK - 1))
        def _fire_next():
            hash_and_fire(g + jnp.int32(1))

        pltpu.make_async_copy(
            tab_hbm.at[pl.ds(0, ROWS)],
            rows_v.at[pl.ds(buf * ROWS, ROWS)], sem_r).wait()

        buf768 = buf * jnp.int32(ROWS)
        buf256 = buf * jnp.int32(CPB)
        iks = [_iota16() + jnp.int32(16 * k) for k in range(4)]
        zero16 = _bcast(jnp.int32(0) * buf)
        wbase0 = (buf768 + jnp.int32(SROWS)) * jnp.int32(DIM)

        @plsc.parallel_loop(jnp.int32(0), jnp.int32(BLK), jnp.int32(1),
                            unroll=2)
        def combine(b):
            accs = [None] * 4
            for c in range(N_CHUNKS):
                bc = b * N_CHUNKS + c
                pk = plsc.load_gather(pk_v, [_bcast(buf256 + bc)])
                off = pk & jnp.int32(0xFFFF)
                wflat = _bcast(wbase0 + bc * jnp.int32(DIM))                     + lax.shift_right_logical(pk, jnp.int32(16))
                wgt = plsc.load_gather(rows_v, [zero16, wflat])
                sbase = _bcast((buf768 + 2 * bc) * jnp.int32(DIM)) + off
                for k in range(4):
                    v = plsc.load_gather(rows_v, [zero16, sbase + iks[k]])
                    vw = v * wgt
                    accs[k] = vw if accs[k] is None else accs[k] + vw
            for k in range(4):
                out_v[b, pl.ds(16 * k, 16)] = accs[k] * jnp.float32(SCALE8)

        base = wid * B_PER_W + g * BLK
        pltpu.sync_copy(out_v, out_hbm.at[pl.ds(base, BLK), :])
        return _

    hash_and_fire(jnp.int32(0))
    lax.fori_loop(jnp.int32(0), jnp.int32(NBLK), block, None)


@jax.jit
def _run(x32, table2, cf32):
    mesh = plsc.VectorSubcoreMesh(core_axis_name="c", subcore_axis_name="s")
    kfn = pl.kernel(
        _body,
        out_type=jax.ShapeDtypeStruct((BATCH, DIM), jnp.float32),
        mesh=mesh,
        scratch_types=[
            pltpu.VMEM((BLK,), jnp.int32),
            pltpu.VMEM((32, 16), jnp.int32),
            pltpu.VMEM((2 * ROWS // 128, 128), jnp.int32),
            pltpu.VMEM((2 * CPB,), jnp.int32),
            pltpu.VMEM((2 * ROWS, DIM), jnp.float32),
            pltpu.VMEM((BLK, DIM), jnp.float32),
            pltpu.SemaphoreType.DMA,
        ],
        compiler_params=pltpu.CompilerParams(
            needs_layout_passes=False, use_tc_tiling_on_sc=False),
    )
    return kfn(x32, table2, cf32)


def kernel(x, table, coeffs0, coeffs1):
    x32 = x.astype(jnp.int32)
    cf32 = jnp.concatenate([
        coeffs0[:, 0], coeffs0[:, 1], coeffs1[:, 0], coeffs1[:, 1],
    ]).astype(jnp.int32)
    cf32 = jnp.tile(cf32[:, None], (1, 16))
    table2 = table.reshape(SIZE // DIM, DIM)
    return _run(x32, table2, cf32)
